# baseline (speedup 1.0000x reference)
import functools

import jax
import jax.numpy as jnp
from jax import lax
from jax.experimental import pallas as pl
from jax.experimental.pallas import tpu as pltpu
from jax.experimental.pallas import tpu_sc as plsc

_VOCAB = 100000
_D = 16
_NSUB = 12
_E = _D * _NSUB
_BASE = 10000.0
_B, _T = 1024, 200
_BT = _B * _T

_PITCH_SIZE = 128
_POS_SIZE = 32
_BAR_ROWS = _T
_AUX_ROWS = _PITCH_SIZE + _POS_SIZE + _BAR_ROWS

_NC = 2
_NSC = 16
_NW = _NC * _NSC
_ROWS_PER_W = _B // _NW

_CA = 112
_CB = 88
_GROUPS = 13


def _aux_body(bias_ref, out_ref):
    rows = lax.broadcasted_iota(jnp.int32, (_AUX_ROWS, _E), 0)
    cols = lax.broadcasted_iota(jnp.int32, (_AUX_ROWS, _E), 1)
    d = cols % _D
    exponent = 2.0 * jnp.floor(d.astype(jnp.float32) / 2.0) / float(_D)
    rate = jnp.exp(exponent * (-jnp.log(jnp.float32(_BASE))))
    is_pos = (rows >= _PITCH_SIZE) & (rows < _PITCH_SIZE + _POS_SIZE)
    is_bar = rows >= _PITCH_SIZE + _POS_SIZE
    val = jnp.where(
        is_bar, rows - (_PITCH_SIZE + _POS_SIZE),
        jnp.where(is_pos, rows - _PITCH_SIZE, rows)).astype(jnp.float32)
    ang = val * rate
    enc = jnp.where(d % 2 == 0, jnp.sin(ang), jnp.cos(ang))
    pitch_b = bias_ref[0:1, :]
    pos_b = bias_ref[1:2, :]
    bar_b = bias_ref[2:3, :]
    bias = jnp.where(is_bar, bar_b, jnp.where(is_pos, pos_b, pitch_b))
    out_ref[:, :] = enc + bias


_TRBLK = 2048
_NTRBLK = -(-_VOCAB // _TRBLK)
_VPAD = _NTRBLK * _TRBLK


def _tr_body(tin_ref, out_ref):
    t = tin_ref[...].T
    half = _TRBLK // 2
    a = t[:, 0:128].reshape(half, 2, 128)[:, 0, :]
    b1 = t[:, 128:192].reshape(half, 2, 64)[:, 0, :]
    b2 = t[:, 0:64].reshape(half, 2, 64)[:, 1, :]
    b = jnp.concatenate([b1, b2], axis=1)
    c = t[:, 64:192].reshape(half, 2, 128)[:, 1, :]
    out_ref[...] = jnp.stack([a, b, c], axis=1).reshape(3 * half, 128)


def _tc_transpose(table_t):
    return pl.pallas_call(
        _tr_body,
        grid=(_NTRBLK,),
        in_specs=[pl.BlockSpec((_E, _TRBLK), lambda i: (0, i))],
        out_specs=pl.BlockSpec((_TRBLK * _E // 128, 128), lambda i: (i, 0)),
        out_shape=jax.ShapeDtypeStruct((_VPAD * _E // 128, 128), jnp.float32),
    )(table_t)


def _build_aux(biases):
    return pl.pallas_call(
        _aux_body,
        out_shape=jax.ShapeDtypeStruct((_AUX_ROWS, _E), jnp.float32),
    )(biases)


_TOK_W = _T * _ROWS_PER_W
_CHUNK = 128
_NCHUNK = _TOK_W // _CHUNK
_NBUF = 4


def _sc_body(idx_hbm, table_hbm, aux_hbm, out_hbm,
             idx_all, rb0, rb1, rb2, rb3, auxbuf,
             gs0, gs1, gs2, gs3, ws0, ws1, ws2, ws3, osem):
    bufs = (rb0, rb1, rb2, rb3)
    gsems = (gs0, gs1, gs2, gs3)
    wsems = (ws0, ws1, ws2, ws3)
    cid = lax.axis_index("c")
    sid = lax.axis_index("s")
    wid = sid * _NC + cid
    lanes = lax.iota(jnp.int32, 16)
    tok0 = pl.multiple_of(wid * _TOK_W, 8)

    pltpu.sync_copy(idx_hbm.at[pl.ds(tok0, _TOK_W)],
                    idx_all.at[pl.ds(0, _TOK_W)])
    idx_all[pl.ds(_TOK_W, 16)] = jnp.zeros((16,), jnp.int32)

    def g_start(c, s):
        pltpu.async_copy(
            table_hbm.at[idx_all.at[pl.ds(c * _CHUNK, _CHUNK)]],
            bufs[s], gsems[s])

    def g_wait(s):
        pltpu.make_async_copy(
            table_hbm.at[idx_all.at[pl.ds(0, _CHUNK)]],
            bufs[s], gsems[s]).wait()

    def w_start(c, s):
        pltpu.async_copy(
            bufs[s], out_hbm.at[pl.ds(tok0 + c * _CHUNK, _CHUNK)], wsems[s])

    def w_wait(s):
        pltpu.make_async_copy(
            bufs[s], out_hbm.at[pl.ds(tok0, _CHUNK)], wsems[s]).wait()

    g_start(0, 0)
    g_start(1, 1)

    def chunk4_body(c4, _):
        for s in range(_NBUF):
            d = c4 * _NBUF + s
            s2 = (s + 2) % _NBUF

            @pl.when(d >= 2)
            def _():
                w_wait(s2)

            @pl.when(d + 2 < _NCHUNK)
            def _():
                g_start(d + 2, s2)

            g_wait(s)
            w_start(d, s)
        return 0

    lax.fori_loop(0, _NCHUNK // _NBUF, chunk4_body, 0)
    for d in range(_NCHUNK - _NCHUNK % _NBUF, _NCHUNK):
        s = d % _NBUF
        s2 = (s + 2) % _NBUF
        w_wait(s2)
        g_wait(s)
        w_start(d, s)
    for d in range(_NCHUNK - 2, _NCHUNK):
        w_wait(d % _NBUF)

    def row_body(r, _):
        rb = pl.multiple_of(r * _T, 8)
        carry = jnp.int32(0)
        for g in range(_GROUPS):
            iv = idx_all[pl.ds(rb + g * 16, 16)]
            valid = lanes < (16 if g < 12 else 8)
            barm = (iv == 128) & valid
            bar_ones = jnp.where(barm, 1, 0)
            bar_idx = carry + jnp.cumsum(bar_ones) - 1
            carry = carry + jnp.sum(bar_ones)
            maskv = (iv < 161) & valid
            auxi = jnp.where(iv == 128, 160 + bar_idx,
                             jnp.where(iv < 128, iv, iv - 1))
            posg = tok0 + rb + g * 16 + lanes
            cnt = jnp.sum(jnp.where(maskv, 1, 0))

            @pl.when(cnt > 0)
            def _():
                lanestar = jnp.max(jnp.where(maskv, lanes, -1))
                onehot = jnp.where(lanes == lanestar, 1, 0)
                astar = jnp.sum(onehot * auxi)
                pstar = jnp.sum(onehot * posg)
                auxf = jnp.where(maskv, auxi, astar)
                posf = jnp.where(maskv, posg, pstar)
                pltpu.async_copy(aux_hbm.at[auxf], auxbuf, osem).wait()
                pltpu.async_copy(auxbuf, out_hbm.at[posf], osem).wait()
        return 0

    lax.fori_loop(0, _ROWS_PER_W, row_body, 0)


_sc_gather = functools.partial(
    pl.kernel,
    mesh=plsc.VectorSubcoreMesh(core_axis_name="c", subcore_axis_name="s"),
    out_type=jax.ShapeDtypeStruct((_BT, _E), jnp.float32),
    compiler_params=pltpu.CompilerParams(
        needs_layout_passes=False, use_tc_tiling_on_sc=False),
    scratch_types=[
        pltpu.VMEM((_TOK_W + 16,), jnp.int32),
        pltpu.VMEM((_CHUNK, _E), jnp.float32),
        pltpu.VMEM((_CHUNK, _E), jnp.float32),
        pltpu.VMEM((_CHUNK, _E), jnp.float32),
        pltpu.VMEM((_CHUNK, _E), jnp.float32),
        pltpu.VMEM((16, _E), jnp.float32),
        pltpu.SemaphoreType.DMA,
        pltpu.SemaphoreType.DMA,
        pltpu.SemaphoreType.DMA,
        pltpu.SemaphoreType.DMA,
        pltpu.SemaphoreType.DMA,
        pltpu.SemaphoreType.DMA,
        pltpu.SemaphoreType.DMA,
        pltpu.SemaphoreType.DMA,
        pltpu.SemaphoreType.DMA,
    ],
)(_sc_body)


def kernel(idx, table, pitch_bias, pos_bias, bar_bias):
    biases = jnp.concatenate([
        pitch_bias.reshape(1, _E),
        pos_bias.reshape(1, _E),
        bar_bias.reshape(1, _E),
    ], axis=0)
    aux = _build_aux(biases)
    table_lin = _tc_transpose(table.T).reshape(_VPAD, _E)
    out = _sc_gather(idx.reshape(_BT).astype(jnp.int32), table_lin, aux)
    return out.reshape(_B, _T, _E)


# speedup vs baseline: 1.0012x; 1.0012x over previous
import functools

import jax
import jax.numpy as jnp
from jax import lax
from jax.experimental import pallas as pl
from jax.experimental.pallas import tpu as pltpu
from jax.experimental.pallas import tpu_sc as plsc

_VOCAB = 100000
_D = 16
_NSUB = 12
_E = _D * _NSUB
_BASE = 10000.0
_B, _T = 1024, 200
_BT = _B * _T

_PITCH_SIZE = 128
_POS_SIZE = 32
_BAR_ROWS = _T
_AUX_ROWS = _PITCH_SIZE + _POS_SIZE + _BAR_ROWS

_NC = 2
_NSC = 16
_NW = _NC * _NSC
_ROWS_PER_W = _B // _NW

_CA = 112
_CB = 88
_GROUPS = 13


def _aux_body(bias_ref, out_ref):
    rows = lax.broadcasted_iota(jnp.int32, (_AUX_ROWS, _E), 0)
    cols = lax.broadcasted_iota(jnp.int32, (_AUX_ROWS, _E), 1)
    d = cols % _D
    exponent = 2.0 * jnp.floor(d.astype(jnp.float32) / 2.0) / float(_D)
    rate = jnp.exp(exponent * (-jnp.log(jnp.float32(_BASE))))
    is_pos = (rows >= _PITCH_SIZE) & (rows < _PITCH_SIZE + _POS_SIZE)
    is_bar = rows >= _PITCH_SIZE + _POS_SIZE
    val = jnp.where(
        is_bar, rows - (_PITCH_SIZE + _POS_SIZE),
        jnp.where(is_pos, rows - _PITCH_SIZE, rows)).astype(jnp.float32)
    ang = val * rate
    enc = jnp.where(d % 2 == 0, jnp.sin(ang), jnp.cos(ang))
    pitch_b = bias_ref[0:1, :]
    pos_b = bias_ref[1:2, :]
    bar_b = bias_ref[2:3, :]
    bias = jnp.where(is_bar, bar_b, jnp.where(is_pos, pos_b, pitch_b))
    out_ref[:, :] = enc + bias


_TRBLK = 2048
_NTRBLK = -(-_VOCAB // _TRBLK)
_VPAD = _NTRBLK * _TRBLK


def _tr_body(tin_ref, out_ref):
    t = tin_ref[...].T
    half = _TRBLK // 2
    a = t[:, 0:128].reshape(half, 2, 128)[:, 0, :]
    b1 = t[:, 128:192].reshape(half, 2, 64)[:, 0, :]
    b2 = t[:, 0:64].reshape(half, 2, 64)[:, 1, :]
    b = jnp.concatenate([b1, b2], axis=1)
    c = t[:, 64:192].reshape(half, 2, 128)[:, 1, :]
    out_ref[...] = jnp.stack([a, b, c], axis=1).reshape(3 * half, 128)


def _tc_transpose(table_t):
    return pl.pallas_call(
        _tr_body,
        grid=(_NTRBLK,),
        in_specs=[pl.BlockSpec((_E, _TRBLK), lambda i: (0, i))],
        out_specs=pl.BlockSpec((_TRBLK * _E // 128, 128), lambda i: (i, 0)),
        out_shape=jax.ShapeDtypeStruct((_VPAD * _E // 128, 128), jnp.float32),
    )(table_t)


def _build_aux(biases):
    return pl.pallas_call(
        _aux_body,
        out_shape=jax.ShapeDtypeStruct((_AUX_ROWS, _E), jnp.float32),
    )(biases)


_TOK_W = _T * _ROWS_PER_W
_CHUNK = 80
_NCHUNK = _TOK_W // _CHUNK
_NBUF = 6
_LOOKAHEAD = 4


def _sc_body(idx_hbm, table_hbm, aux_hbm, out_hbm,
             idx_all, rb0, rb1, rb2, rb3, rb4, rb5, auxbuf,
             gs0, gs1, gs2, gs3, gs4, gs5,
             ws0, ws1, ws2, ws3, ws4, ws5, osem):
    bufs = (rb0, rb1, rb2, rb3, rb4, rb5)
    gsems = (gs0, gs1, gs2, gs3, gs4, gs5)
    wsems = (ws0, ws1, ws2, ws3, ws4, ws5)
    cid = lax.axis_index("c")
    sid = lax.axis_index("s")
    wid = sid * _NC + cid
    lanes = lax.iota(jnp.int32, 16)
    tok0 = pl.multiple_of(wid * _TOK_W, 8)

    pltpu.sync_copy(idx_hbm.at[pl.ds(tok0, _TOK_W)],
                    idx_all.at[pl.ds(0, _TOK_W)])
    idx_all[pl.ds(_TOK_W, 16)] = jnp.zeros((16,), jnp.int32)

    def g_start(c, s):
        pltpu.async_copy(
            table_hbm.at[idx_all.at[pl.ds(c * _CHUNK, _CHUNK)]],
            bufs[s], gsems[s])

    def g_wait(s):
        pltpu.make_async_copy(
            table_hbm.at[idx_all.at[pl.ds(0, _CHUNK)]],
            bufs[s], gsems[s]).wait()

    def w_start(c, s):
        pltpu.async_copy(
            bufs[s], out_hbm.at[pl.ds(tok0 + c * _CHUNK, _CHUNK)], wsems[s])

    def w_wait(s):
        pltpu.make_async_copy(
            bufs[s], out_hbm.at[pl.ds(tok0, _CHUNK)], wsems[s]).wait()

    for p in range(_LOOKAHEAD):
        g_start(p, p)

    def chunk_body(cb, _):
        for s in range(_NBUF):
            d = cb * _NBUF + s
            sg = (s + _LOOKAHEAD) % _NBUF

            @pl.when(d + _LOOKAHEAD < _NCHUNK)
            def _():
                @pl.when(d >= _NBUF - _LOOKAHEAD)
                def _():
                    w_wait(sg)

                g_start(d + _LOOKAHEAD, sg)

            g_wait(s)
            w_start(d, s)
        return 0

    lax.fori_loop(0, _NCHUNK // _NBUF, chunk_body, 0)
    for d in range(_NCHUNK - _NCHUNK % _NBUF, _NCHUNK):
        s = d % _NBUF
        g_wait(s)
        w_start(d, s)
    for d in range(_NCHUNK - _NBUF, _NCHUNK):
        w_wait(d % _NBUF)

    def row_body(r, _):
        rb = pl.multiple_of(r * _T, 8)
        carry = jnp.int32(0)
        for g in range(_GROUPS):
            iv = idx_all[pl.ds(rb + g * 16, 16)]
            valid = lanes < (16 if g < 12 else 8)
            barm = (iv == 128) & valid
            bar_ones = jnp.where(barm, 1, 0)
            bar_idx = carry + jnp.cumsum(bar_ones) - 1
            carry = carry + jnp.sum(bar_ones)
            maskv = (iv < 161) & valid
            auxi = jnp.where(iv == 128, 160 + bar_idx,
                             jnp.where(iv < 128, iv, iv - 1))
            posg = tok0 + rb + g * 16 + lanes
            cnt = jnp.sum(jnp.where(maskv, 1, 0))

            @pl.when(cnt > 0)
            def _():
                lanestar = jnp.max(jnp.where(maskv, lanes, -1))
                onehot = jnp.where(lanes == lanestar, 1, 0)
                astar = jnp.sum(onehot * auxi)
                pstar = jnp.sum(onehot * posg)
                auxf = jnp.where(maskv, auxi, astar)
                posf = jnp.where(maskv, posg, pstar)
                pltpu.async_copy(aux_hbm.at[auxf], auxbuf, osem).wait()
                pltpu.async_copy(auxbuf, out_hbm.at[posf], osem).wait()
        return 0

    lax.fori_loop(0, _ROWS_PER_W, row_body, 0)


_sc_gather = functools.partial(
    pl.kernel,
    mesh=plsc.VectorSubcoreMesh(core_axis_name="c", subcore_axis_name="s"),
    out_type=jax.ShapeDtypeStruct((_BT, _E), jnp.float32),
    compiler_params=pltpu.CompilerParams(
        needs_layout_passes=False, use_tc_tiling_on_sc=False),
    scratch_types=(
        [pltpu.VMEM((_TOK_W + 16,), jnp.int32)]
        + [pltpu.VMEM((_CHUNK, _E), jnp.float32) for _ in range(_NBUF)]
        + [pltpu.VMEM((16, _E), jnp.float32)]
        + [pltpu.SemaphoreType.DMA for _ in range(2 * _NBUF + 1)]
    ),
)(_sc_body)


def kernel(idx, table, pitch_bias, pos_bias, bar_bias):
    biases = jnp.concatenate([
        pitch_bias.reshape(1, _E),
        pos_bias.reshape(1, _E),
        bar_bias.reshape(1, _E),
    ], axis=0)
    aux = _build_aux(biases)
    table_lin = _tc_transpose(table.T).reshape(_VPAD, _E)
    out = _sc_gather(idx.reshape(_BT).astype(jnp.int32), table_lin, aux)
    return out.reshape(_B, _T, _E)


# speedup vs baseline: 1.0137x; 1.0124x over previous
import functools

import jax
import jax.numpy as jnp
from jax import lax
from jax.experimental import pallas as pl
from jax.experimental.pallas import tpu as pltpu
from jax.experimental.pallas import tpu_sc as plsc

_VOCAB = 100000
_D = 16
_NSUB = 12
_E = _D * _NSUB
_BASE = 10000.0
_B, _T = 1024, 200
_BT = _B * _T

_PITCH_SIZE = 128
_POS_SIZE = 32
_BAR_ROWS = _T
_AUX_ROWS = _PITCH_SIZE + _POS_SIZE + _BAR_ROWS

_NC = 2
_NSC = 16
_NW = _NC * _NSC
_ROWS_PER_W = _B // _NW

_CA = 112
_CB = 88
_GROUPS = 13


def _aux_body(bias_ref, out_ref):
    rows = lax.broadcasted_iota(jnp.int32, (_AUX_ROWS, _E), 0)
    cols = lax.broadcasted_iota(jnp.int32, (_AUX_ROWS, _E), 1)
    d = cols % _D
    exponent = 2.0 * jnp.floor(d.astype(jnp.float32) / 2.0) / float(_D)
    rate = jnp.exp(exponent * (-jnp.log(jnp.float32(_BASE))))
    is_pos = (rows >= _PITCH_SIZE) & (rows < _PITCH_SIZE + _POS_SIZE)
    is_bar = rows >= _PITCH_SIZE + _POS_SIZE
    val = jnp.where(
        is_bar, rows - (_PITCH_SIZE + _POS_SIZE),
        jnp.where(is_pos, rows - _PITCH_SIZE, rows)).astype(jnp.float32)
    ang = val * rate
    enc = jnp.where(d % 2 == 0, jnp.sin(ang), jnp.cos(ang))
    pitch_b = bias_ref[0:1, :]
    pos_b = bias_ref[1:2, :]
    bar_b = bias_ref[2:3, :]
    bias = jnp.where(is_bar, bar_b, jnp.where(is_pos, pos_b, pitch_b))
    out_ref[:, :] = enc + bias


_TRBLK = 2048
_NTRBLK = -(-_VOCAB // _TRBLK)
_VPAD = _NTRBLK * _TRBLK


def _tr_body(tin_ref, out_ref):
    t = tin_ref[...].T
    half = _TRBLK // 2
    a = t[:, 0:128].reshape(half, 2, 128)[:, 0, :]
    b1 = t[:, 128:192].reshape(half, 2, 64)[:, 0, :]
    b2 = t[:, 0:64].reshape(half, 2, 64)[:, 1, :]
    b = jnp.concatenate([b1, b2], axis=1)
    c = t[:, 64:192].reshape(half, 2, 128)[:, 1, :]
    out_ref[...] = jnp.stack([a, b, c], axis=1).reshape(3 * half, 128)


def _tc_transpose(table_t):
    return pl.pallas_call(
        _tr_body,
        grid=(_NTRBLK,),
        in_specs=[pl.BlockSpec((_E, _TRBLK), lambda i: (0, i))],
        out_specs=pl.BlockSpec((_TRBLK * _E // 128, 128), lambda i: (i, 0)),
        out_shape=jax.ShapeDtypeStruct((_VPAD * _E // 128, 128), jnp.float32),
    )(table_t)


def _build_aux(biases):
    return pl.pallas_call(
        _aux_body,
        out_shape=jax.ShapeDtypeStruct((_AUX_ROWS, _E), jnp.float32),
    )(biases)


_TOK_W = _T * _ROWS_PER_W
_CHUNK = 80
_NCHUNK = _TOK_W // _CHUNK
_NBUF = 6
_LOOKAHEAD = 4


def _sc_body(idx_hbm, table_hbm, aux_hbm, out_hbm,
             idx_all, rb0, rb1, rb2, rb3, rb4, rb5, auxbuf,
             gs0, gs1, gs2, gs3, gs4, gs5,
             ws0, ws1, ws2, ws3, ws4, ws5, osem):
    bufs = (rb0, rb1, rb2, rb3, rb4, rb5)
    gsems = (gs0, gs1, gs2, gs3, gs4, gs5)
    wsems = (ws0, ws1, ws2, ws3, ws4, ws5)
    cid = lax.axis_index("c")
    sid = lax.axis_index("s")
    wid = sid * _NC + cid
    lanes = lax.iota(jnp.int32, 16)
    tok0 = pl.multiple_of(wid * _TOK_W, 8)

    pltpu.sync_copy(idx_hbm.at[pl.ds(tok0, _TOK_W)],
                    idx_all.at[pl.ds(0, _TOK_W)])
    idx_all[pl.ds(_TOK_W, 16)] = jnp.zeros((16,), jnp.int32)

    def g_start(c, s):
        pltpu.async_copy(
            table_hbm.at[idx_all.at[pl.ds(c * _CHUNK, _CHUNK)]],
            bufs[s], gsems[s])

    def g_wait(s):
        pltpu.make_async_copy(
            table_hbm.at[idx_all.at[pl.ds(0, _CHUNK)]],
            bufs[s], gsems[s]).wait()

    def w_start(c, s):
        pltpu.async_copy(
            bufs[s], out_hbm.at[pl.ds(tok0 + c * _CHUNK, _CHUNK)], wsems[s])

    def w_wait(s):
        pltpu.make_async_copy(
            bufs[s], out_hbm.at[pl.ds(tok0, _CHUNK)], wsems[s]).wait()

    def do_row(r):
        rb = pl.multiple_of(r * _T, 8)
        carry = jnp.int32(0)
        for g in range(_GROUPS):
            iv = idx_all[pl.ds(rb + g * 16, 16)]
            valid = lanes < (16 if g < 12 else 8)
            barm = (iv == 128) & valid
            bar_ones = jnp.where(barm, 1, 0)
            bar_idx = carry + jnp.cumsum(bar_ones) - 1
            carry = carry + jnp.sum(bar_ones)
            maskv = (iv < 161) & valid
            auxi = jnp.where(iv == 128, 160 + bar_idx,
                             jnp.where(iv < 128, iv, iv - 1))
            posg = tok0 + rb + g * 16 + lanes
            cnt = jnp.sum(jnp.where(maskv, 1, 0))

            @pl.when(cnt > 0)
            def _():
                lanestar = jnp.max(jnp.where(maskv, lanes, -1))
                onehot = jnp.where(lanes == lanestar, 1, 0)
                astar = jnp.sum(onehot * auxi)
                pstar = jnp.sum(onehot * posg)
                auxf = jnp.where(maskv, auxi, astar)
                posf = jnp.where(maskv, posg, pstar)
                pltpu.async_copy(aux_hbm.at[auxf], auxbuf, osem).wait()
                pltpu.async_copy(auxbuf, out_hbm.at[posf], osem).wait()

    for p in range(_LOOKAHEAD):
        g_start(p, p)

    def chunk_body(cb, next_row):
        for s in range(_NBUF):
            d = cb * _NBUF + s
            sg = (s + _LOOKAHEAD) % _NBUF

            @pl.when(d + _LOOKAHEAD < _NCHUNK)
            def _():
                @pl.when(d >= _NBUF - _LOOKAHEAD)
                def _():
                    w_wait(sg)

                g_start(d + _LOOKAHEAD, sg)

            g_wait(s)
            w_start(d, s)
        avail = jnp.minimum(
            jnp.int32(_ROWS_PER_W),
            (_CHUNK * (cb * _NBUF + _NBUF - 2)) // _T)
        for _ in range(3):
            nr = next_row

            @pl.when(nr < avail)
            def _():
                do_row(nr)

            next_row = jnp.where(nr < avail, nr + 1, nr)
        return next_row

    next_row = lax.fori_loop(0, _NCHUNK // _NBUF, chunk_body, jnp.int32(0))
    for d in range(_NCHUNK - _NCHUNK % _NBUF, _NCHUNK):
        s = d % _NBUF
        g_wait(s)
        w_start(d, s)
    for d in range(_NCHUNK - _NBUF, _NCHUNK):
        w_wait(d % _NBUF)

    def row_body(r, _):
        do_row(r)
        return 0

    lax.fori_loop(next_row, _ROWS_PER_W, row_body, 0)


_sc_gather = functools.partial(
    pl.kernel,
    mesh=plsc.VectorSubcoreMesh(core_axis_name="c", subcore_axis_name="s"),
    out_type=jax.ShapeDtypeStruct((_BT, _E), jnp.float32),
    compiler_params=pltpu.CompilerParams(
        needs_layout_passes=False, use_tc_tiling_on_sc=False),
    scratch_types=(
        [pltpu.VMEM((_TOK_W + 16,), jnp.int32)]
        + [pltpu.VMEM((_CHUNK, _E), jnp.float32) for _ in range(_NBUF)]
        + [pltpu.VMEM((16, _E), jnp.float32)]
        + [pltpu.SemaphoreType.DMA for _ in range(2 * _NBUF + 1)]
    ),
)(_sc_body)


def kernel(idx, table, pitch_bias, pos_bias, bar_bias):
    biases = jnp.concatenate([
        pitch_bias.reshape(1, _E),
        pos_bias.reshape(1, _E),
        bar_bias.reshape(1, _E),
    ], axis=0)
    aux = _build_aux(biases)
    table_lin = _tc_transpose(table.T).reshape(_VPAD, _E)
    out = _sc_gather(idx.reshape(_BT).astype(jnp.int32), table_lin, aux)
    return out.reshape(_B, _T, _E)
